# baseline (speedup 1.0000x reference)
import functools

import jax
import jax.numpy as jnp
from jax import lax
from jax.experimental import pallas as pl
from jax.experimental.pallas import tpu as pltpu
from jax.experimental.pallas import tpu_sc as plsc

_info = plsc.get_sparse_core_info()
_NC, _NS, _L = _info.num_cores, _info.num_subcores, _info.num_lanes
_NW = _NC * _NS


def _make_sc_score(batch, dim):
  half = dim // 2
  bpw = batch // _NW
  hb = bpw // 2
  gph = hb // _L
  mesh = plsc.VectorSubcoreMesh(core_axis_name="c", subcore_axis_name="s")

  @functools.partial(
      pl.kernel,
      mesh=mesh,
      out_type=jax.ShapeDtypeStruct((batch,), jnp.float32),
      compiler_params=pltpu.CompilerParams(needs_layout_passes=False),
      scratch_types=[
          pltpu.VMEM((bpw,), jnp.int32),
          pltpu.VMEM((bpw,), jnp.int32),
          pltpu.VMEM((bpw,), jnp.int32),
          pltpu.VMEM((bpw, dim), jnp.float32),
          pltpu.VMEM((bpw, dim), jnp.float32),
          pltpu.VMEM((bpw, dim), jnp.float32),
          pltpu.VMEM((bpw,), jnp.float32),
          pltpu.SemaphoreType.DMA,
          pltpu.SemaphoreType.DMA,
          pltpu.SemaphoreType.DMA,
          pltpu.SemaphoreType.DMA,
          pltpu.SemaphoreType.DMA,
      ],
  )
  def sc_score(hidx_hbm, ridx_hbm, tidx_hbm, ent_hbm, rel_hbm, out_hbm,
               hidx_v, ridx_v, tidx_v, hrow_v, rrow_v, trow_v, out_v,
               semh, semr, semt, sem0, sem1):
    wid = lax.axis_index("s") * _NC + lax.axis_index("c")
    base = wid * bpw
    sl_all = pl.ds(base, bpw)
    ch_i = pltpu.async_copy(hidx_hbm.at[sl_all], hidx_v, semh)
    cr_i = pltpu.async_copy(ridx_hbm.at[sl_all], ridx_v, semr)
    ct_i = pltpu.async_copy(tidx_hbm.at[sl_all], tidx_v, semt)

    sems = (sem0, sem1)
    sls = tuple(pl.ds(h * hb, hb) for h in range(2))
    ch_i.wait()
    gh = [pltpu.async_copy(ent_hbm.at[hidx_v.at[sls[h]]], hrow_v.at[sls[h]],
                           sems[h]) for h in range(2)]
    cr_i.wait()
    gr = [pltpu.async_copy(rel_hbm.at[ridx_v.at[sls[h]]], rrow_v.at[sls[h]],
                           sems[h]) for h in range(2)]
    ct_i.wait()
    gt = [pltpu.async_copy(ent_hbm.at[tidx_v.at[sls[h]]], trow_v.at[sls[h]],
                           sems[h]) for h in range(2)]
    copies = [(gh[h], gr[h], gt[h]) for h in range(2)]

    lane = lax.iota(jnp.int32, _L)

    for h in range(2):
      for c in copies[h]:
        c.wait()
      rows = [h * hb + g * _L + lane for g in range(gph)]

      def dim_body(d, accs, rows=rows):
        rot = (lane + d) & (half - 1)
        im_col = rot + half
        out = []
        for g in range(gph):
          rh = plsc.load_gather(hrow_v, [rows[g], rot])
          ih = plsc.load_gather(hrow_v, [rows[g], im_col])
          rr = plsc.load_gather(rrow_v, [rows[g], rot])
          ir = plsc.load_gather(rrow_v, [rows[g], im_col])
          rt = plsc.load_gather(trow_v, [rows[g], rot])
          it = plsc.load_gather(trow_v, [rows[g], im_col])
          out.append(accs[g] + (rh * rr - ih * ir) * rt
                     + (rh * ir + ih * rr) * it)
        return tuple(out)

      accs = lax.fori_loop(
          0, half, dim_body,
          tuple(jnp.zeros((_L,), jnp.float32) for _ in range(gph)),
          unroll=4)
      for g in range(gph):
        out_v[pl.ds(h * hb + g * _L, _L)] = accs[g]

    pltpu.sync_copy(out_v, out_hbm.at[pl.ds(base, bpw)])

  return sc_score


def kernel(sample, entity_embedding, relation_embedding):
  batch = sample.shape[0]
  dim = entity_embedding.shape[1]
  hidx = sample[:, 0]
  ridx = sample[:, 1]
  tidx = sample[:, 2]
  score = _make_sc_score(batch, dim)(
      hidx, ridx, tidx, entity_embedding, relation_embedding)
  return score.reshape(batch, 1)


# speedup vs baseline: 1.1552x; 1.1552x over previous
import functools

import jax
import jax.numpy as jnp
from jax import lax
from jax.experimental import pallas as pl
from jax.experimental.pallas import tpu as pltpu
from jax.experimental.pallas import tpu_sc as plsc

_info = plsc.get_sparse_core_info()
_NC, _NS, _L = _info.num_cores, _info.num_subcores, _info.num_lanes
_NW = _NC * _NS


def _make_sc_score(batch, dim):
  half = dim // 2
  bpw = batch // _NW
  hb = bpw // 2
  gph = hb // _L
  mesh = plsc.VectorSubcoreMesh(core_axis_name="c", subcore_axis_name="s")

  @functools.partial(
      pl.kernel,
      mesh=mesh,
      out_type=jax.ShapeDtypeStruct((batch,), jnp.float32),
      compiler_params=pltpu.CompilerParams(needs_layout_passes=False),
      scratch_types=[
          pltpu.VMEM((bpw,), jnp.int32),
          pltpu.VMEM((bpw,), jnp.int32),
          pltpu.VMEM((bpw,), jnp.int32),
          pltpu.VMEM((bpw, dim), jnp.float32),
          pltpu.VMEM((bpw, dim), jnp.float32),
          pltpu.VMEM((bpw, dim), jnp.float32),
          pltpu.VMEM((bpw,), jnp.float32),
          pltpu.SemaphoreType.DMA,
          pltpu.SemaphoreType.DMA,
          pltpu.SemaphoreType.DMA,
          pltpu.SemaphoreType.DMA,
          pltpu.SemaphoreType.DMA,
      ],
  )
  def sc_score(hidx_hbm, ridx_hbm, tidx_hbm, ent_hbm, rel_hbm, out_hbm,
               hidx_v, ridx_v, tidx_v, hrow_v, rrow_v, trow_v, out_v,
               semh, semr, semt, sem0, sem1):
    wid = lax.axis_index("s") * _NC + lax.axis_index("c")
    base = wid * bpw
    sl_all = pl.ds(base, bpw)
    ch_i = pltpu.async_copy(hidx_hbm.at[sl_all], hidx_v, semh)
    cr_i = pltpu.async_copy(ridx_hbm.at[sl_all], ridx_v, semr)
    ct_i = pltpu.async_copy(tidx_hbm.at[sl_all], tidx_v, semt)

    sems = (sem0, sem1)
    sls = tuple(pl.ds(h * hb, hb) for h in range(2))
    ch_i.wait()
    gh = [pltpu.async_copy(ent_hbm.at[hidx_v.at[sls[h]]], hrow_v.at[sls[h]],
                           sems[h]) for h in range(2)]
    cr_i.wait()
    gr = [pltpu.async_copy(rel_hbm.at[ridx_v.at[sls[h]]], rrow_v.at[sls[h]],
                           sems[h]) for h in range(2)]
    ct_i.wait()
    gt = [pltpu.async_copy(ent_hbm.at[tidx_v.at[sls[h]]], trow_v.at[sls[h]],
                           sems[h]) for h in range(2)]
    copies = [(gh[h], gr[h], gt[h]) for h in range(2)]

    lane = lax.iota(jnp.int32, _L)

    for h in range(2):
      for c in copies[h]:
        c.wait()
      rows = [h * hb + g * _L + lane for g in range(gph)]

      def dim_body(d, accs, rows=rows):
        rot = (lane + d) & (half - 1)
        im_col = rot + half
        out = []
        for g in range(gph):
          rh = plsc.load_gather(hrow_v, [rows[g], rot])
          ih = plsc.load_gather(hrow_v, [rows[g], im_col])
          rr = plsc.load_gather(rrow_v, [rows[g], rot])
          ir = plsc.load_gather(rrow_v, [rows[g], im_col])
          rt = plsc.load_gather(trow_v, [rows[g], rot])
          it = plsc.load_gather(trow_v, [rows[g], im_col])
          out.append(accs[g] + (rh * rr - ih * ir) * rt
                     + (rh * ir + ih * rr) * it)
        return tuple(out)

      accs = lax.fori_loop(
          0, half, dim_body,
          tuple(jnp.zeros((_L,), jnp.float32) for _ in range(gph)))
      for g in range(gph):
        out_v[pl.ds(h * hb + g * _L, _L)] = accs[g]

    pltpu.sync_copy(out_v, out_hbm.at[pl.ds(base, bpw)])

  return sc_score


def kernel(sample, entity_embedding, relation_embedding):
  batch = sample.shape[0]
  dim = entity_embedding.shape[1]
  hidx = sample[:, 0]
  ridx = sample[:, 1]
  tidx = sample[:, 2]
  score = _make_sc_score(batch, dim)(
      hidx, ridx, tidx, entity_embedding, relation_embedding)
  return score.reshape(batch, 1)
